# baseline (speedup 1.0000x reference)
import functools

import jax
import jax.numpy as jnp
from jax import lax
from jax.experimental import pallas as pl
from jax.experimental.pallas import tpu as pltpu
from jax.experimental.pallas import tpu_sc as plsc

_VOCAB = 100000
_D = 768
_N_TOK = 10
_B = 4
_S = 2048

_NC = 2
_NS = 16
_NW = _NC * _NS

_ROWS = _B * _S
_RPW = _ROWS // _NW
_CHUNK = 64
_NCHUNK = _RPW // _CHUNK

_mesh = plsc.VectorSubcoreMesh(core_axis_name="c", subcore_axis_name="s")


@functools.partial(
    pl.kernel,
    mesh=_mesh,
    compiler_params=pltpu.CompilerParams(use_tc_tiling_on_sc=False),
    out_type=jax.ShapeDtypeStruct((_ROWS, _D), jnp.float32),
    scratch_types=[
        pltpu.VMEM((_NCHUNK, _CHUNK), jnp.int32),
        pltpu.VMEM((2, _CHUNK, _D), jnp.float32),
        pltpu.VMEM((_N_TOK, _D), jnp.float32),
        pltpu.SemaphoreType.DMA,
    ],
)
def _soft_embed(tokens_hbm, wte_hbm, learned_hbm, out_hbm, idx_v, rows_v,
                learned_v, gsem):
    wid = lax.axis_index("s") * _NC + lax.axis_index("c")
    base = wid * _RPW

    pltpu.sync_copy(tokens_hbm.at[wid], idx_v)

    copies = [None, None]
    copies[0] = pltpu.async_copy(wte_hbm.at[idx_v.at[0]], rows_v.at[0], gsem)
    for j in range(_NCHUNK):
        cur = j % 2
        if j + 1 < _NCHUNK:
            copies[(j + 1) % 2] = pltpu.async_copy(
                wte_hbm.at[idx_v.at[j + 1]], rows_v.at[(j + 1) % 2], gsem
            )
        copies[cur].wait()
        pltpu.sync_copy(rows_v.at[cur], out_hbm.at[pl.ds(base + j * _CHUNK, _CHUNK)])
        if j == 0:
            @pl.when(base % _S == 0)
            def _():
                pltpu.sync_copy(learned_hbm, learned_v)
                pltpu.sync_copy(learned_v, out_hbm.at[pl.ds(base, _N_TOK)])


def kernel(tokens, wte, learned_embedding):
    tok = tokens.reshape(_NW, _NCHUNK, _CHUNK)
    out = _soft_embed(tok, wte, learned_embedding)
    return out.reshape(_B, _S, _D)


# speedup vs baseline: 8.7219x; 8.7219x over previous
import functools

import jax
import jax.numpy as jnp
from jax import lax
from jax.experimental import pallas as pl
from jax.experimental.pallas import tpu as pltpu
from jax.experimental.pallas import tpu_sc as plsc

_VOCAB = 100000
_D = 768
_N_TOK = 10
_B = 4
_S = 2048

_NC = 2
_NS = 16
_NW = _NC * _NS
_L = 16

_ROWS = _B * _S
_RPW = _ROWS // _NW
_BLK = 64
_NBLK = _RPW // _BLK
_BURST = _BLK // _L
_WPB = _S // _RPW

_mesh = plsc.VectorSubcoreMesh(core_axis_name="c", subcore_axis_name="s")


@functools.partial(
    pl.kernel,
    mesh=_mesh,
    out_type=jax.ShapeDtypeStruct((_ROWS, _D), jnp.float32),
    scratch_types=[
        pltpu.VMEM((_RPW,), jnp.int32),
        pltpu.VMEM((2, _BLK, _D), jnp.float32),
        pltpu.VMEM((_L, _D), jnp.float32),
        pltpu.SemaphoreType.DMA,
        pltpu.SemaphoreType.DMA,
        pltpu.SemaphoreType.DMA,
    ],
)
def _soft_embed(tokens_hbm, wte_hbm, learned_hbm, out_hbm,
                idx_v, rows_v, learned_v, gsem, osem, lsem):
    wid = lax.axis_index("s") * _NC + lax.axis_index("c")
    base = wid * _RPW
    batch_start = base % _S == 0

    pltpu.sync_copy(tokens_hbm.at[pl.ds(base, _RPW)], idx_v)

    @pl.when(batch_start)
    def _():
        pltpu.sync_copy(learned_hbm, learned_v)

    def fire(p, buf):
        ds = []
        for k in range(_BURST):
            vidx = idx_v[pl.ds(p * _BLK + k * _L, _L)]
            ds.append(pltpu.async_copy(
                wte_hbm.at[vidx], rows_v.at[buf].at[pl.ds(k * _L, _L)], gsem))
        return ds

    gds = [None, None]
    wds = [None, None]
    gds[0] = fire(0, 0)
    for p in range(_NBLK):
        cur = p % 2
        nxt = (p + 1) % 2
        if p + 1 < _NBLK:
            if wds[nxt] is not None:
                wds[nxt].wait()
                wds[nxt] = None
            gds[nxt] = fire(p + 1, nxt)
        for d in gds[cur]:
            d.wait()
        wds[cur] = pltpu.async_copy(
            rows_v.at[cur], out_hbm.at[pl.ds(base + p * _BLK, _BLK)], osem)
    for w in wds:
        if w is not None:
            w.wait()

    @pl.when(batch_start)
    def _():
        iota = lax.iota(jnp.int32, _L)
        svidx = jnp.minimum(iota, _N_TOK - 1) + (wid // _WPB) * _S
        pltpu.async_copy(learned_v, out_hbm.at[svidx], lsem).wait()


def kernel(tokens, wte, learned_embedding):
    tok = tokens.reshape(_ROWS)
    learned_pad = jnp.concatenate(
        [learned_embedding,
         jnp.broadcast_to(learned_embedding[_N_TOK - 1:_N_TOK],
                          (_L - _N_TOK, _D))], axis=0)
    out = _soft_embed(tok, wte, learned_pad)
    return out.reshape(_B, _S, _D)
